# Initial kernel scaffold; baseline (speedup 1.0000x reference)
import jax
import jax.numpy as jnp
from jax.experimental import pallas as pl


def kernel(x, edge_index, edge_weight, W1, W2):
    raise NotImplementedError("write your pallas kernel here")


# speedup vs baseline: 3.6363x; 3.6363x over previous
import functools

import jax
import jax.numpy as jnp
from jax import lax
from jax.experimental import pallas as pl
from jax.experimental.pallas import tpu as pltpu
from jax.experimental.pallas import tpu_sc as plsc

NC = 2
NS = 16
CH = 80
RB = 400


def _bcast_lane(v16, j):
  idx = jnp.full((16, 1), j, dtype=jnp.int32)
  dnums = lax.GatherDimensionNumbers(
      offset_dims=(), collapsed_slice_dims=(0,), start_index_map=(0,))
  return lax.gather(v16, idx, dnums, slice_sizes=(1,),
                    mode=lax.GatherScatterMode.PROMISE_IN_BOUNDS)


def _make_spmm(n_rows, n_tab, D, EPW, col_split):
  NCHUNK = EPW // CH
  RPSF = (-(-n_rows // NS) + 7) // 8 * 8
  RPSL = n_rows - (NS - 1) * RPSF
  assert RPSL > 0
  G16 = CH // 16
  DV = D // 16
  mesh = plsc.VectorSubcoreMesh(core_axis_name="c", subcore_axis_name="s",
                                num_cores=NC, num_subcores=NS)

  @functools.partial(
      pl.kernel,
      out_type=jax.ShapeDtypeStruct((2 * n_rows, D), jnp.float32),
      mesh=mesh,
      scratch_types=[
          pltpu.VMEM_SHARED((n_rows, D), jnp.float32),
          pltpu.VMEM((EPW,), jnp.int32),
          pltpu.VMEM((EPW,), jnp.int32),
          pltpu.VMEM((EPW,), jnp.float32),
          pltpu.VMEM((CH, D), jnp.float32),
          pltpu.VMEM((CH,), jnp.int32),
          pltpu.SemaphoreType.DMA,
          pltpu.SemaphoreType.DMA,
      ],
  )
  def spmm(tab_hbm, src_hbm, dst_hbm, w_hbm, zer_hbm, out_hbm,
           accum, src_v, dst_v, w_v, rows, dstc, gsem, zsem):
    c = lax.axis_index("c")
    s = lax.axis_index("s")
    if col_split:
      sl = s
    else:
      sl = s * NC + c
    ebase = pl.multiple_of(sl * EPW, 8)
    rbase = pl.multiple_of(s * RPSF, 8)

    @pl.when(s < NS - 1)
    def _():
      pltpu.async_copy(zer_hbm, accum.at[pl.ds(rbase, RPSF)], zsem).wait()

    @pl.when(s == NS - 1)
    def _():
      pltpu.async_copy(zer_hbm.at[pl.ds(0, RPSL)],
                       accum.at[pl.ds(rbase, RPSL)], zsem).wait()

    pltpu.sync_copy(src_hbm.at[pl.ds(ebase, EPW)], src_v)
    pltpu.sync_copy(dst_hbm.at[pl.ds(ebase, EPW)], dst_v)
    pltpu.sync_copy(w_hbm.at[pl.ds(ebase, EPW)], w_v)
    if col_split:
      off = c * n_rows

      def addoff(i, carry):
        o = pl.ds(pl.multiple_of(i * 16, 16), 16)
        src_v[o] = src_v[o] + off
        return carry

      lax.fori_loop(0, EPW // 16, addoff, 0)
    plsc.subcore_barrier()

    def chunk_body(g, carry):
      gb = pl.multiple_of(g * CH, CH)
      pltpu.async_copy(tab_hbm.at[src_v.at[pl.ds(gb, CH)]], rows, gsem).wait()
      for grp in range(G16):
        o = pl.ds(pl.multiple_of(gb + grp * 16, 16), 16)
        dstc[pl.ds(grp * 16, 16)] = dst_v[o]
        w16 = w_v[o]
        for j in range(16):
          wj = _bcast_lane(w16, j)
          e = grp * 16 + j
          for k in range(DV):
            csl = pl.ds(k * 16, 16)
            rows[e, csl] = rows[e, csl] * wj
      pltpu.sync_copy(rows, accum.at[dstc], add=True)
      return carry

    lax.fori_loop(0, NCHUNK, chunk_body, 0)
    plsc.subcore_barrier()

    obase = pl.multiple_of(c * n_rows + rbase, 8)

    @pl.when(s < NS - 1)
    def _():
      pltpu.sync_copy(accum.at[pl.ds(rbase, RPSF)],
                      out_hbm.at[pl.ds(obase, RPSF)])

    @pl.when(s == NS - 1)
    def _():
      pltpu.sync_copy(accum.at[pl.ds(rbase, RPSL)],
                      out_hbm.at[pl.ds(obase, RPSL)])

  return spmm


def _mm1(x, W1, n):
  nb = n // RB

  def body(x_ref, w_ref, o_ref):
    o_ref[...] = jnp.dot(x_ref[...], w_ref[...],
                         preferred_element_type=jnp.float32)

  return pl.pallas_call(
      body,
      grid=(NC, nb),
      in_specs=[
          pl.BlockSpec((RB, x.shape[1]), lambda c, i: (i, 0)),
          pl.BlockSpec((W1.shape[0], 128), lambda c, i: (0, c)),
      ],
      out_specs=pl.BlockSpec((RB, 128), lambda c, i, _nb=nb: (c * _nb + i, 0)),
      out_shape=jax.ShapeDtypeStruct((2 * n, 128), jnp.float32),
  )(x, W1)


def _mm2(h1, W2p, n):
  nb = n // RB

  def body(a_ref, b_ref, w_ref, o_ref):
    w = w_ref[...]
    a = jnp.maximum(a_ref[...], 0.0)
    b = jnp.maximum(b_ref[...], 0.0)
    o_ref[...] = (
        jnp.dot(a, w[:128], preferred_element_type=jnp.float32)
        + jnp.dot(b, w[128:], preferred_element_type=jnp.float32))

  return pl.pallas_call(
      body,
      grid=(nb,),
      in_specs=[
          pl.BlockSpec((RB, 128), lambda i: (i, 0)),
          pl.BlockSpec((RB, 128), lambda i, _nb=nb: (_nb + i, 0)),
          pl.BlockSpec(W2p.shape, lambda i: (0, 0)),
      ],
      out_specs=pl.BlockSpec((RB, 128), lambda i: (i, 0)),
      out_shape=jax.ShapeDtypeStruct((n, 128), jnp.float32),
  )(h1, h1, W2p)


def _finish(p, n, ncls):
  nb = n // RB

  def body(a_ref, b_ref, o_ref):
    z = jnp.maximum(a_ref[:, :ncls] + b_ref[:, :ncls], 0.0)
    z = z - jnp.max(z, axis=1, keepdims=True)
    o_ref[...] = z - jnp.log(jnp.sum(jnp.exp(z), axis=1, keepdims=True))

  return pl.pallas_call(
      body,
      grid=(nb,),
      in_specs=[
          pl.BlockSpec((RB, 128), lambda i: (i, 0)),
          pl.BlockSpec((RB, 128), lambda i, _nb=nb: (_nb + i, 0)),
      ],
      out_specs=pl.BlockSpec((RB, ncls), lambda i: (i, 0)),
      out_shape=jax.ShapeDtypeStruct((n, ncls), jnp.float32),
  )(p, p)


@jax.jit
def kernel(x, edge_index, edge_weight, W1, W2):
  n = x.shape[0]
  ncls = W2.shape[1]
  e = edge_weight.shape[0]

  quant = NC * NS * CH
  ep = ((e + quant - 1) // quant) * quant
  pad = ep - e
  src = jnp.concatenate([edge_index[1], jnp.zeros((pad,), jnp.int32)])
  dst = jnp.concatenate([edge_index[0], jnp.zeros((pad,), jnp.int32)])
  w = jnp.concatenate([edge_weight, jnp.zeros((pad,), jnp.float32)])

  rpsf = (-(-n // NS) + 7) // 8 * 8
  zer = jnp.zeros((rpsf, 128), jnp.float32)
  W2p = jnp.pad(W2, ((0, 0), (0, 128 - ncls)))

  sup1 = _mm1(x, W1, n)
  spmm1 = _make_spmm(n, 2 * n, 128, ep // NS, col_split=True)
  h1 = spmm1(sup1, src, dst, w, zer)
  sup2 = _mm2(h1, W2p, n)
  spmm2 = _make_spmm(n, n, 128, ep // (NC * NS), col_split=False)
  p = spmm2(sup2, src, dst, w, zer)
  return _finish(p, n, ncls)
